# Initial kernel scaffold; baseline (speedup 1.0000x reference)
import jax
import jax.numpy as jnp
from jax.experimental import pallas as pl


def kernel(x, edge, W1, b1, W2, b2, Wg1, bg1, Wg2, bg2, Wo, bo):
    raise NotImplementedError("write your pallas kernel here")


# speedup vs baseline: 2.5663x; 2.5663x over previous
import jax
import jax.numpy as jnp
from jax.experimental import pallas as pl

N = 100000
BN = 2000


def _out_mm_kernel(x_ref, w_ref, b_ref, o_ref):
    o_ref[...] = jax.nn.relu(
        jnp.dot(x_ref[...], w_ref[...], preferred_element_type=jnp.float32)
        + b_ref[...]
    )


def _gcn(h, src, dst, dinv, W, b):
    hw = h @ W.T
    msg = (dinv[:, None] * hw)[src]
    agg = jnp.zeros_like(hw).at[dst].add(msg)
    return dinv[:, None] * agg + dinv[:, None] ** 2 * hw + b


def kernel(x, edge, W1, b1, W2, b2, Wg1, bg1, Wg2, bg2, Wo, bo):
    cur = x[-1]
    src, dst = edge[0], edge[1]
    deg = jnp.zeros((N,), jnp.float32).at[dst].add(1.0) + 1.0
    dinv = jax.lax.rsqrt(deg)
    tx = jax.nn.relu(cur @ W1.T + b1)
    tx = jax.nn.relu(tx @ W2.T + b2)
    stx1 = jax.nn.relu(_gcn(tx, src, dst, dinv, Wg1, bg1) + tx)
    stx2 = _gcn(stx1, src, dst, dinv, Wg2, bg2)
    out = pl.pallas_call(
        _out_mm_kernel,
        grid=(N // BN,),
        in_specs=[
            pl.BlockSpec((BN, 32), lambda i: (i, 0)),
            pl.BlockSpec((32, 128), lambda i: (0, 0)),
            pl.BlockSpec((1, 128), lambda i: (0, 0)),
        ],
        out_specs=pl.BlockSpec((BN, 128), lambda i: (i, 0)),
        out_shape=jax.ShapeDtypeStruct((N, 128), jnp.float32),
    )(stx2, Wo.T, bo.reshape(1, 128))
    return out


# speedup vs baseline: 5.8569x; 2.2822x over previous
import functools

import jax
import jax.numpy as jnp
from jax import lax
from jax.experimental import pallas as pl
from jax.experimental.pallas import tpu as pltpu
from jax.experimental.pallas import tpu_sc as plsc

N = 100000
E = 1600000
HALF = N // 2
AGG_ROWS = 50048
DUMMY = 50008
QUART = N // 4
AGG_Q = 25088
DUMMY_Q = 25024
NT = 16
CHUNK = 1024
K = 128
NCH = 98
E_PAD = NT * NCH * CHUNK
ROWS_PT = NCH * (CHUNK // K)

BN = 2000


def _sc_mesh():
    return plsc.VectorSubcoreMesh(core_axis_name="c", subcore_axis_name="s")


@functools.partial(
    pl.kernel,
    out_type=jax.ShapeDtypeStruct((4, AGG_Q, 32), jnp.float32),
    mesh=_sc_mesh(),
    compiler_params=pltpu.CompilerParams(use_tc_tiling_on_sc=False),
    scratch_types=[
        pltpu.VMEM((CHUNK // K, K), jnp.int32),
        pltpu.VMEM((CHUNK // K, K), jnp.int32),
        pltpu.VMEM((CHUNK, 32), jnp.float32),
        pltpu.VMEM_SHARED((AGG_Q, 32), jnp.float32),
        pltpu.SemaphoreType.DMA,
        pltpu.SemaphoreType.DMA,
    ],
)
def _sc_agg(src_hbm, dstq_hbm, g_hbm, z_hbm, out_hbm,
            sidx, didx, rows, agg, gsem, ssem):
    cid = lax.axis_index("c")
    sid = lax.axis_index("s")
    zrows = AGG_Q // NT
    row0 = sid * ROWS_PT

    for p in range(2):
        q = cid * 2 + p
        pltpu.sync_copy(z_hbm, agg.at[pl.ds(sid * zrows, zrows)])
        plsc.subcore_barrier()

        def chunk(c, carry):
            rb = row0 + c * (CHUNK // K)
            pltpu.sync_copy(src_hbm.at[pl.ds(rb, CHUNK // K)], sidx)
            pltpu.sync_copy(dstq_hbm.at[q].at[pl.ds(rb, CHUNK // K)], didx)
            gathers = [
                pltpu.async_copy(g_hbm.at[sidx.at[j]],
                                 rows.at[pl.ds(j * K, K)], gsem)
                for j in range(CHUNK // K)
            ]
            for d in gathers:
                d.wait()
            scatters = [
                pltpu.async_copy(rows.at[pl.ds(j * K, K)],
                                 agg.at[didx.at[j]], ssem, add=True)
                for j in range(CHUNK // K)
            ]
            for d in scatters:
                d.wait()
            return carry

        lax.fori_loop(0, NCH, chunk, 0)
        plsc.subcore_barrier()
        pltpu.sync_copy(agg.at[pl.ds(sid * zrows, zrows)],
                        out_hbm.at[q].at[pl.ds(sid * zrows, zrows)])


@functools.partial(
    pl.kernel,
    out_type=jax.ShapeDtypeStruct((2, AGG_ROWS, 16), jnp.float32),
    mesh=_sc_mesh(),
    compiler_params=pltpu.CompilerParams(use_tc_tiling_on_sc=False),
    scratch_types=[
        pltpu.VMEM((CHUNK // K, K), jnp.int32),
        pltpu.VMEM((CHUNK, 16), jnp.float32),
        pltpu.VMEM_SHARED((AGG_ROWS, 16), jnp.float32),
        pltpu.SemaphoreType.DMA,
    ],
)
def _sc_deg(dstb_hbm, ones_hbm, z_hbm, out_hbm, didx, ones_v, deg, ssem):
    cid = lax.axis_index("c")
    sid = lax.axis_index("s")
    zrows = AGG_ROWS // NT
    pltpu.sync_copy(z_hbm, deg.at[pl.ds(sid * zrows, zrows)])
    pltpu.sync_copy(ones_hbm, ones_v)
    plsc.subcore_barrier()

    row0 = sid * ROWS_PT

    def chunk(c, carry):
        rb = row0 + c * (CHUNK // K)
        pltpu.sync_copy(dstb_hbm.at[cid].at[pl.ds(rb, CHUNK // K)], didx)
        scatters = [
            pltpu.async_copy(ones_v.at[pl.ds(j * K, K)],
                             deg.at[didx.at[j]], ssem, add=True)
            for j in range(CHUNK // K)
        ]
        for d in scatters:
            d.wait()
        return carry

    lax.fori_loop(0, NCH, chunk, 0)
    plsc.subcore_barrier()
    pltpu.sync_copy(deg.at[pl.ds(sid * zrows, zrows)],
                    out_hbm.at[cid].at[pl.ds(sid * zrows, zrows)])


def _tc_pre(cur_ref, deg_ref, w1_ref, b1_ref, w2_ref, b2_ref, wg1_ref,
            tx_ref, g1_ref, dinv_ref):
    dinv = lax.rsqrt(deg_ref[...] + 1.0)
    t = jax.nn.relu(
        jnp.dot(cur_ref[...], w1_ref[...],
                preferred_element_type=jnp.float32) + b1_ref[...])
    t = jax.nn.relu(
        jnp.dot(t, w2_ref[...], preferred_element_type=jnp.float32)
        + b2_ref[...])
    tx_ref[...] = t
    hw1 = jnp.dot(t, wg1_ref[...], preferred_element_type=jnp.float32)
    g1_ref[...] = dinv * hw1
    dinv_ref[...] = dinv


def _tc_mid(agg1_ref, tx_ref, dinv_ref, wg1_ref, bg1_ref, wg2_ref,
            stx1_ref, g2_ref):
    dinv = dinv_ref[...]
    tx = tx_ref[...]
    hw1 = jnp.dot(tx, wg1_ref[...], preferred_element_type=jnp.float32)
    s1 = jax.nn.relu(dinv * agg1_ref[...] + dinv * dinv * hw1
                     + bg1_ref[...] + tx)
    stx1_ref[...] = s1
    g2_ref[...] = dinv * jnp.dot(s1, wg2_ref[...],
                                 preferred_element_type=jnp.float32)


def _tc_fin(agg2_ref, stx1_ref, dinv_ref, wg2_ref, bg2_ref, wo_ref, bo_ref,
            out_ref):
    dinv = dinv_ref[...]
    s1 = stx1_ref[...]
    hw2 = jnp.dot(s1, wg2_ref[...], preferred_element_type=jnp.float32)
    s2 = dinv * agg2_ref[...] + dinv * dinv * hw2 + bg2_ref[...]
    out_ref[...] = jax.nn.relu(
        jnp.dot(s2, wo_ref[...], preferred_element_type=jnp.float32)
        + bo_ref[...])


def _row_spec(w):
    return pl.BlockSpec((BN, w), lambda i: (i, 0))


def _full_spec(r, w):
    return pl.BlockSpec((r, w), lambda i: (0, 0))


def kernel(x, edge, W1, b1, W2, b2, Wg1, bg1, Wg2, bg2, Wo, bo):
    cur = x[-1]
    src = edge[0]
    dst = edge[1]

    pad = E_PAD - E
    src_p = jnp.pad(src, (0, pad)).reshape(E_PAD // K, K)
    dst_p = jnp.pad(dst, (0, pad), constant_values=-1)
    dst0 = jnp.where((dst_p >= 0) & (dst_p < HALF), dst_p, DUMMY)
    dst1 = jnp.where(dst_p >= HALF, dst_p - HALF, DUMMY)
    dstb = jnp.stack([dst0, dst1]).reshape(2, E_PAD // K, K)
    dstq = jnp.stack([
        jnp.where((dst_p >= q * QUART) & (dst_p < (q + 1) * QUART),
                  dst_p - q * QUART, DUMMY_Q)
        for q in range(4)
    ]).reshape(4, E_PAD // K, K)

    z32 = jnp.zeros((AGG_Q // NT, 32), jnp.float32)
    z1 = jnp.zeros((AGG_ROWS // NT, 16), jnp.float32)
    ones = jnp.ones((CHUNK, 16), jnp.float32)

    deg = _sc_deg(dstb, ones, z1)[:, :HALF, :1].reshape(N, 1)

    tx, g1, dinv = pl.pallas_call(
        _tc_pre,
        grid=(N // BN,),
        in_specs=[
            _row_spec(128), _row_spec(1), _full_spec(128, 32),
            _full_spec(1, 32), _full_spec(32, 32), _full_spec(1, 32),
            _full_spec(32, 32),
        ],
        out_specs=[_row_spec(32), _row_spec(32), _row_spec(1)],
        out_shape=[
            jax.ShapeDtypeStruct((N, 32), jnp.float32),
            jax.ShapeDtypeStruct((N, 32), jnp.float32),
            jax.ShapeDtypeStruct((N, 1), jnp.float32),
        ],
    )(cur, deg, W1.T, b1.reshape(1, 32), W2.T, b2.reshape(1, 32), Wg1.T)

    agg1 = _sc_agg(src_p, dstq, g1, z32)[:, :QUART].reshape(N, 32)

    stx1, g2 = pl.pallas_call(
        _tc_mid,
        grid=(N // BN,),
        in_specs=[
            _row_spec(32), _row_spec(32), _row_spec(1),
            _full_spec(32, 32), _full_spec(1, 32), _full_spec(32, 32),
        ],
        out_specs=[_row_spec(32), _row_spec(32)],
        out_shape=[
            jax.ShapeDtypeStruct((N, 32), jnp.float32),
            jax.ShapeDtypeStruct((N, 32), jnp.float32),
        ],
    )(agg1, tx, dinv, Wg1.T, bg1.reshape(1, 32), Wg2.T)

    agg2 = _sc_agg(src_p, dstq, g2, z32)[:, :QUART].reshape(N, 32)

    out = pl.pallas_call(
        _tc_fin,
        grid=(N // BN,),
        in_specs=[
            _row_spec(32), _row_spec(32), _row_spec(1),
            _full_spec(32, 32), _full_spec(1, 32), _full_spec(32, 128),
            _full_spec(1, 128),
        ],
        out_specs=_row_spec(128),
        out_shape=jax.ShapeDtypeStruct((N, 128), jnp.float32),
    )(agg2, stx1, dinv, Wg2.T, bg2.reshape(1, 32), Wo.T, bo.reshape(1, 128))
    return out
